# baseline (speedup 1.0000x reference)
import functools

import jax
import jax.numpy as jnp
from jax import lax
from jax.experimental import pallas as pl
from jax.experimental.pallas import tpu as pltpu
from jax.experimental.pallas import tpu_sc as plsc

NODES = 50000
DIM = 64
EDGES = 800000

_NC = 2
_NS = 16
_NW = _NC * _NS
_EPW = EDGES // _NW
_C = 128
_NFULL = _EPW // _C
_REM = _EPW - _NFULL * _C
_NPAIR = (_NFULL - 1) // 2


@functools.partial(
    pl.kernel,
    out_type=jax.ShapeDtypeStruct((EDGES,), jnp.float32),
    mesh=plsc.VectorSubcoreMesh(core_axis_name="c", subcore_axis_name="s"),
    compiler_params=pltpu.CompilerParams(use_tc_tiling_on_sc=False),
    scratch_types=[
        pltpu.VMEM((_EPW,), jnp.int32),
        pltpu.VMEM((_EPW,), jnp.int32),
        pltpu.VMEM((_C, DIM // 2), jnp.int32),
        pltpu.VMEM((_C, DIM // 2), jnp.int32),
        pltpu.VMEM((_C, DIM // 2), jnp.int32),
        pltpu.VMEM((_C, DIM // 2), jnp.int32),
        pltpu.VMEM((_EPW,), jnp.float32),
        pltpu.SemaphoreType.DMA,
        pltpu.SemaphoreType.DMA,
        pltpu.SemaphoreType.DMA,
        pltpu.SemaphoreType.DMA,
    ],
)
def _edge_dot(src_hbm, dst_hbm, w_hbm, out_hbm,
              idx_s, idx_d, rs0, rd0, rs1, rd1, out_v,
              ss0, sd0, ss1, sd1):
    wid = lax.axis_index("s") * _NC + lax.axis_index("c")
    base0 = wid * _EPW

    pltpu.sync_copy(src_hbm.at[pl.ds(base0, _EPW)], idx_s)
    pltpu.sync_copy(dst_hbm.at[pl.ds(base0, _EPW)], idx_d)

    lane = lax.iota(jnp.int32, 16)
    rot_idx = [((lane + (1 << k)) & 15).reshape(16, 1) for k in range(4)]
    _dnums = lax.GatherDimensionNumbers(
        offset_dims=(), collapsed_slice_dims=(0,), start_index_map=(0,))

    def hsum(p):
        for k in range(4):
            p = p + lax.gather(
                p, rot_idx[k], _dnums, (1,),
                mode=lax.GatherScatterMode.PROMISE_IN_BOUNDS)
        return p

    def start(lb, n, bs, bd, ss, sd):
        pltpu.async_copy(
            w_hbm.at[idx_s.at[pl.ds(lb, n)]], bs.at[pl.ds(0, n)], ss)
        pltpu.async_copy(
            w_hbm.at[idx_d.at[pl.ds(lb, n)]], bd.at[pl.ds(0, n)], sd)

    def wait(n, bs, bd, ss, sd):
        pltpu.make_async_copy(
            w_hbm.at[idx_s.at[pl.ds(0, n)]], bs.at[pl.ds(0, n)], ss).wait()
        pltpu.make_async_copy(
            w_hbm.at[idx_d.at[pl.ds(0, n)]], bd.at[pl.ds(0, n)], sd).wait()

    def compute(local_base, bs, bd, ngroups, tail):
        def unpack2(w):
            lo = lax.bitcast_convert_type(w << 16, jnp.float32)
            hi = lax.bitcast_convert_type(w & jnp.int32(-65536), jnp.float32)
            return lo, hi

        def edge_total(e):
            a0l, a0h = unpack2(bs[e, pl.ds(0, 16)])
            b0l, b0h = unpack2(bd[e, pl.ds(0, 16)])
            a1l, a1h = unpack2(bs[e, pl.ds(16, 16)])
            b1l, b1h = unpack2(bd[e, pl.ds(16, 16)])
            p = a0l * b0l + a0h * b0h
            p += a1l * b1l + a1h * b1h
            return hsum(p)

        def do_group(start_e):
            res = jnp.zeros((16,), jnp.float32)
            for l in range(16):
                res = jnp.where(lane == l, edge_total(start_e + l), res)
            out_v[pl.ds(local_base + start_e, 16)] = res

        lax.fori_loop(0, ngroups, lambda g, _: (do_group(g * 16), _)[1], None)
        if tail:
            do_group(ngroups * 16 + tail - 16)

    start(0, _C, rs0, rd0, ss0, sd0)

    def pair_body(k, _):
        c0 = (2 * k) * _C
        start(c0 + _C, _C, rs1, rd1, ss1, sd1)
        wait(_C, rs0, rd0, ss0, sd0)
        compute(c0, rs0, rd0, _C // 16, 0)
        start(c0 + 2 * _C, _C, rs0, rd0, ss0, sd0)
        wait(_C, rs1, rd1, ss1, sd1)
        compute(c0 + _C, rs1, rd1, _C // 16, 0)
        return _

    lax.fori_loop(0, _NPAIR, pair_body, None)

    last_full = (_NFULL - 1) * _C
    start(_NFULL * _C, _REM, rs1, rd1, ss1, sd1)
    wait(_C, rs0, rd0, ss0, sd0)
    compute(last_full, rs0, rd0, _C // 16, 0)
    wait(_REM, rs1, rd1, ss1, sd1)
    compute(_NFULL * _C, rs1, rd1, _REM // 16, _REM % 16)

    pltpu.sync_copy(out_v, out_hbm.at[pl.ds(base0, _EPW)])


def kernel(edge_label_index, weight):
    src = edge_label_index[0]
    dst = edge_label_index[1]
    wpacked = lax.bitcast_convert_type(
        weight.astype(jnp.bfloat16).reshape(NODES, DIM // 2, 2), jnp.int32)
    return _edge_dot(src, dst, wpacked)


# speedup vs baseline: 1.3187x; 1.3187x over previous
import functools

import jax
import jax.numpy as jnp
from jax import lax
from jax.experimental import pallas as pl
from jax.experimental.pallas import tpu as pltpu
from jax.experimental.pallas import tpu_sc as plsc

NODES = 50000
DIM = 64
EDGES = 800000

_NC = 2
_NS = 16
_NW = _NC * _NS
_EPW = EDGES // _NW
_C = 128
_NFULL = _EPW // _C
_REM = _EPW - _NFULL * _C
_NPAIR = (_NFULL - 1) // 2

_PACK_ROWS = 4096


def _pack_body(w_ref, out_ref):
    x = w_ref[...]
    rne = x + jnp.int32(0x7FFF) + ((x >> 16) & jnp.int32(1))
    lo = lax.shift_right_logical(rne[:, :DIM // 2], 16)
    hi = rne[:, DIM // 2:] & jnp.int32(-65536)
    out_ref[...] = lo | hi


def _pack_table(wbits):
    grid = (NODES + _PACK_ROWS - 1) // _PACK_ROWS
    return pl.pallas_call(
        _pack_body,
        grid=(grid,),
        in_specs=[pl.BlockSpec((_PACK_ROWS, DIM), lambda i: (i, 0))],
        out_specs=pl.BlockSpec((_PACK_ROWS, DIM // 2), lambda i: (i, 0)),
        out_shape=jax.ShapeDtypeStruct((NODES, DIM // 2), jnp.int32),
    )(wbits)


@functools.partial(
    pl.kernel,
    out_type=jax.ShapeDtypeStruct((EDGES,), jnp.float32),
    mesh=plsc.VectorSubcoreMesh(core_axis_name="c", subcore_axis_name="s"),
    compiler_params=pltpu.CompilerParams(use_tc_tiling_on_sc=False),
    scratch_types=[
        pltpu.VMEM((_EPW,), jnp.int32),
        pltpu.VMEM((_EPW,), jnp.int32),
        pltpu.VMEM((_C, DIM // 2), jnp.int32),
        pltpu.VMEM((_C, DIM // 2), jnp.int32),
        pltpu.VMEM((_C, DIM // 2), jnp.int32),
        pltpu.VMEM((_C, DIM // 2), jnp.int32),
        pltpu.VMEM((_EPW,), jnp.float32),
        pltpu.SemaphoreType.DMA,
        pltpu.SemaphoreType.DMA,
        pltpu.SemaphoreType.DMA,
        pltpu.SemaphoreType.DMA,
    ],
)
def _edge_dot(eli_hbm, w_hbm, out_hbm,
              idx_s, idx_d, rs0, rd0, rs1, rd1, out_v,
              ss0, sd0, ss1, sd1):
    wid = lax.axis_index("s") * _NC + lax.axis_index("c")
    base0 = wid * _EPW

    pltpu.sync_copy(eli_hbm.at[pl.ds(base0, _EPW)], idx_s)
    pltpu.sync_copy(eli_hbm.at[pl.ds(EDGES + base0, _EPW)], idx_d)

    lane = lax.iota(jnp.int32, 16)
    rot_idx = [((lane + (1 << k)) & 15).reshape(16, 1) for k in range(4)]
    _dnums = lax.GatherDimensionNumbers(
        offset_dims=(), collapsed_slice_dims=(0,), start_index_map=(0,))

    def hsum(p):
        for k in range(4):
            p = p + lax.gather(
                p, rot_idx[k], _dnums, (1,),
                mode=lax.GatherScatterMode.PROMISE_IN_BOUNDS)
        return p

    def start(lb, n, bs, bd, ss, sd):
        pltpu.async_copy(
            w_hbm.at[idx_s.at[pl.ds(lb, n)]], bs.at[pl.ds(0, n)], ss)
        pltpu.async_copy(
            w_hbm.at[idx_d.at[pl.ds(lb, n)]], bd.at[pl.ds(0, n)], sd)

    def wait(n, bs, bd, ss, sd):
        pltpu.make_async_copy(
            w_hbm.at[idx_s.at[pl.ds(0, n)]], bs.at[pl.ds(0, n)], ss).wait()
        pltpu.make_async_copy(
            w_hbm.at[idx_d.at[pl.ds(0, n)]], bd.at[pl.ds(0, n)], sd).wait()

    def compute(local_base, bs, bd, ngroups, tail):
        def unpack2(w):
            lo = lax.bitcast_convert_type(w << 16, jnp.float32)
            hi = lax.bitcast_convert_type(w & jnp.int32(-65536), jnp.float32)
            return lo, hi

        def edge_total(e):
            a0l, a0h = unpack2(bs[e, pl.ds(0, 16)])
            b0l, b0h = unpack2(bd[e, pl.ds(0, 16)])
            a1l, a1h = unpack2(bs[e, pl.ds(16, 16)])
            b1l, b1h = unpack2(bd[e, pl.ds(16, 16)])
            p = a0l * b0l + a0h * b0h
            p += a1l * b1l + a1h * b1h
            return hsum(p)

        def do_group(start_e):
            res = jnp.zeros((16,), jnp.float32)
            for l in range(16):
                res = jnp.where(lane == l, edge_total(start_e + l), res)
            out_v[pl.ds(local_base + start_e, 16)] = res

        lax.fori_loop(0, ngroups, lambda g, _: (do_group(g * 16), _)[1], None)
        if tail:
            do_group(ngroups * 16 + tail - 16)

    start(0, _C, rs0, rd0, ss0, sd0)

    def pair_body(k, _):
        c0 = (2 * k) * _C
        start(c0 + _C, _C, rs1, rd1, ss1, sd1)
        wait(_C, rs0, rd0, ss0, sd0)
        compute(c0, rs0, rd0, _C // 16, 0)
        start(c0 + 2 * _C, _C, rs0, rd0, ss0, sd0)
        wait(_C, rs1, rd1, ss1, sd1)
        compute(c0 + _C, rs1, rd1, _C // 16, 0)
        return _

    lax.fori_loop(0, _NPAIR, pair_body, None)

    last_full = (_NFULL - 1) * _C
    start(_NFULL * _C, _REM, rs1, rd1, ss1, sd1)
    wait(_C, rs0, rd0, ss0, sd0)
    compute(last_full, rs0, rd0, _C // 16, 0)
    wait(_REM, rs1, rd1, ss1, sd1)
    compute(_NFULL * _C, rs1, rd1, _REM // 16, _REM % 16)

    pltpu.sync_copy(out_v, out_hbm.at[pl.ds(base0, _EPW)])


def kernel(edge_label_index, weight):
    eli_flat = edge_label_index.reshape(-1)
    wbits = lax.bitcast_convert_type(weight, jnp.int32)
    wpacked = _pack_table(wbits)
    return _edge_dot(eli_flat, wpacked)


# speedup vs baseline: 1.3769x; 1.0442x over previous
import functools

import jax
import jax.numpy as jnp
from jax import lax
from jax.experimental import pallas as pl
from jax.experimental.pallas import tpu as pltpu
from jax.experimental.pallas import tpu_sc as plsc

NODES = 50000
DIM = 64
EDGES = 800000

_NC = 2
_NS = 16
_NW = _NC * _NS
_EPW = EDGES // _NW
_C = 128
_NFULL = _EPW // _C
_REM = _EPW - _NFULL * _C
_NPAIR = (_NFULL - 1) // 2

_PACK_ROWS = 4096


def _pack_body(w_ref, out_ref):
    x = lax.bitcast_convert_type(w_ref[...], jnp.int32)
    rne = x + jnp.int32(0x7FFF) + ((x >> 16) & jnp.int32(1))
    lo = lax.shift_right_logical(rne[:, :DIM // 2], 16)
    hi = rne[:, DIM // 2:] & jnp.int32(-65536)
    out_ref[...] = lo | hi


def _pack_table(weight):
    grid = (NODES + _PACK_ROWS - 1) // _PACK_ROWS
    return pl.pallas_call(
        _pack_body,
        grid=(grid,),
        in_specs=[pl.BlockSpec((_PACK_ROWS, DIM), lambda i: (i, 0))],
        out_specs=pl.BlockSpec((_PACK_ROWS, DIM // 2), lambda i: (i, 0)),
        out_shape=jax.ShapeDtypeStruct((NODES, DIM // 2), jnp.int32),
    )(weight)


@functools.partial(
    pl.kernel,
    out_type=jax.ShapeDtypeStruct((EDGES,), jnp.float32),
    mesh=plsc.VectorSubcoreMesh(core_axis_name="c", subcore_axis_name="s"),
    compiler_params=pltpu.CompilerParams(use_tc_tiling_on_sc=False),
    scratch_types=[
        pltpu.VMEM((_EPW,), jnp.int32),
        pltpu.VMEM((_EPW,), jnp.int32),
        pltpu.VMEM((_C, DIM // 2), jnp.int32),
        pltpu.VMEM((_C, DIM // 2), jnp.int32),
        pltpu.VMEM((_C, DIM // 2), jnp.int32),
        pltpu.VMEM((_C, DIM // 2), jnp.int32),
        pltpu.VMEM((_EPW,), jnp.float32),
        pltpu.SemaphoreType.DMA,
        pltpu.SemaphoreType.DMA,
        pltpu.SemaphoreType.DMA,
        pltpu.SemaphoreType.DMA,
    ],
)
def _edge_dot(eli_hbm, w_hbm, out_hbm,
              idx_s, idx_d, rs0, rd0, rs1, rd1, out_v,
              ss0, sd0, ss1, sd1):
    wid = lax.axis_index("s") * _NC + lax.axis_index("c")
    base0 = wid * _EPW

    pltpu.sync_copy(eli_hbm.at[0, pl.ds(base0, _EPW)], idx_s)
    pltpu.sync_copy(eli_hbm.at[1, pl.ds(base0, _EPW)], idx_d)

    lane = lax.iota(jnp.int32, 16)
    rot_idx = [((lane + (1 << k)) & 15).reshape(16, 1) for k in range(4)]
    _dnums = lax.GatherDimensionNumbers(
        offset_dims=(), collapsed_slice_dims=(0,), start_index_map=(0,))

    def hsum(p):
        for k in range(4):
            p = p + lax.gather(
                p, rot_idx[k], _dnums, (1,),
                mode=lax.GatherScatterMode.PROMISE_IN_BOUNDS)
        return p

    def start(lb, n, bs, bd, ss, sd):
        pltpu.async_copy(
            w_hbm.at[idx_s.at[pl.ds(lb, n)]], bs.at[pl.ds(0, n)], ss)
        pltpu.async_copy(
            w_hbm.at[idx_d.at[pl.ds(lb, n)]], bd.at[pl.ds(0, n)], sd)

    def wait(n, bs, bd, ss, sd):
        pltpu.make_async_copy(
            w_hbm.at[idx_s.at[pl.ds(0, n)]], bs.at[pl.ds(0, n)], ss).wait()
        pltpu.make_async_copy(
            w_hbm.at[idx_d.at[pl.ds(0, n)]], bd.at[pl.ds(0, n)], sd).wait()

    def compute(local_base, bs, bd, ngroups, tail):
        def unpack2(w):
            lo = lax.bitcast_convert_type(w << 16, jnp.float32)
            hi = lax.bitcast_convert_type(w & jnp.int32(-65536), jnp.float32)
            return lo, hi

        def edge_total(e):
            a0l, a0h = unpack2(bs[e, pl.ds(0, 16)])
            b0l, b0h = unpack2(bd[e, pl.ds(0, 16)])
            a1l, a1h = unpack2(bs[e, pl.ds(16, 16)])
            b1l, b1h = unpack2(bd[e, pl.ds(16, 16)])
            p = a0l * b0l + a0h * b0h
            p += a1l * b1l + a1h * b1h
            return hsum(p)

        def do_group(start_e):
            res = jnp.zeros((16,), jnp.float32)
            for l in range(16):
                res = jnp.where(lane == l, edge_total(start_e + l), res)
            out_v[pl.ds(local_base + start_e, 16)] = res

        lax.fori_loop(0, ngroups, lambda g, _: (do_group(g * 16), _)[1], None)
        if tail:
            do_group(ngroups * 16 + tail - 16)

    start(0, _C, rs0, rd0, ss0, sd0)

    def pair_body(k, _):
        c0 = (2 * k) * _C
        start(c0 + _C, _C, rs1, rd1, ss1, sd1)
        wait(_C, rs0, rd0, ss0, sd0)
        compute(c0, rs0, rd0, _C // 16, 0)
        start(c0 + 2 * _C, _C, rs0, rd0, ss0, sd0)
        wait(_C, rs1, rd1, ss1, sd1)
        compute(c0 + _C, rs1, rd1, _C // 16, 0)
        return _

    lax.fori_loop(0, _NPAIR, pair_body, None)

    last_full = (_NFULL - 1) * _C
    start(_NFULL * _C, _REM, rs1, rd1, ss1, sd1)
    wait(_C, rs0, rd0, ss0, sd0)
    compute(last_full, rs0, rd0, _C // 16, 0)
    wait(_REM, rs1, rd1, ss1, sd1)
    compute(_NFULL * _C, rs1, rd1, _REM // 16, _REM % 16)

    pltpu.sync_copy(out_v, out_hbm.at[pl.ds(base0, _EPW)])


def kernel(edge_label_index, weight):
    wpacked = _pack_table(weight)
    return _edge_dot(edge_label_index, wpacked)
